# baseline (speedup 1.0000x reference)
import functools

import jax
import jax.numpy as jnp
from jax import lax
from jax.experimental import pallas as pl
from jax.experimental.pallas import tpu as pltpu
from jax.experimental.pallas import tpu_sc as plsc

B = 16384
F = 26
V = 100000
D = 16
NUM = 13
C = F * D
OUT_H = C + NUM

QB = B // 8
NQ = B // QB
UNROLL = 4


def _sc_body(
    tab, xcat, xnum, out, sp_out, sp_idx, t_row, idx_v, out_v,
    sem_in, sem_out, sem_idx, sem_ov
):
    cid = lax.axis_index("c")
    sid = lax.axis_index("s")

    pltpu.async_copy(tab.at[cid, sid, :], t_row, sem_in)

    @pl.when(sid == 1)
    def _():
        pltpu.sync_copy(xcat.at[pl.ds(0, 2), :], sp_idx)

    out_desc = None
    for g in range(F // 2):
        f = 2 * g + cid

        pltpu.make_async_copy(tab.at[cid, sid, :], t_row, sem_in).wait()
        if out_desc is not None:
            @pl.when(sid == 0)
            def _():
                pltpu.make_async_copy(
                    sp_out, out.at[pl.ds(0, D), :], sem_out
                ).wait()

        plsc.subcore_barrier()

        pltpu.async_copy(sp_idx.at[cid, pl.ds(0, QB)], idx_v.at[0], sem_idx)
        pltpu.async_copy(sp_idx.at[cid, pl.ds(QB, QB)], idx_v.at[1], sem_idx)

        def pair_body(k, carry):
            q0 = 2 * k
            for s_ in range(2):
                q = q0 + s_
                pltpu.make_async_copy(
                    sp_idx.at[cid, pl.ds(q * QB, QB)], idx_v.at[s_], sem_idx
                ).wait()

                @pl.when(q >= 2)
                def _():
                    pltpu.make_async_copy(
                        out_v.at[s_],
                        sp_out.at[sid, pl.ds(((q - 2) % NQ) * QB, QB)],
                        sem_ov,
                    ).wait()

                def gather_body(j, carry2):
                    for u in range(UNROLL):
                        o = j * (16 * UNROLL) + u * 16
                        iv = idx_v[s_, pl.ds(o, 16)]
                        out_v[s_, pl.ds(o, 16)] = plsc.load_gather(t_row, [iv])
                    return carry2

                lax.fori_loop(0, QB // (16 * UNROLL), gather_body, 0)

                nxt = q + 2

                @pl.when(nxt < NQ)
                def _():
                    pltpu.async_copy(
                        sp_idx.at[cid, pl.ds((nxt % NQ) * QB, QB)],
                        idx_v.at[s_],
                        sem_idx,
                    )

                pltpu.async_copy(
                    out_v.at[s_], sp_out.at[sid, pl.ds(q * QB, QB)], sem_ov
                )
            return carry

        lax.fori_loop(0, NQ // 2, pair_body, 0)

        for q in range(NQ - 2, NQ):
            pltpu.make_async_copy(
                out_v.at[q % 2], sp_out.at[sid, pl.ds(q * QB, QB)], sem_ov
            ).wait()

        if g + 1 < F // 2:
            pltpu.async_copy(tab.at[2 * (g + 1) + cid, sid, :], t_row, sem_in)

        plsc.subcore_barrier()

        if g + 1 < F // 2:
            @pl.when(sid == 1)
            def _():
                pltpu.sync_copy(xcat.at[pl.ds(2 * (g + 1), 2), :], sp_idx)

        @pl.when(sid == 0)
        def _():
            pltpu.async_copy(sp_out, out.at[pl.ds(f * D, D), :], sem_out)
        out_desc = True

    @pl.when(sid == 0)
    def _():
        pltpu.make_async_copy(sp_out, out.at[pl.ds(0, D), :], sem_out).wait()

    @pl.when((sid == 0) & (cid == 0))
    def _():
        pltpu.sync_copy(xnum.at[pl.ds(0, 8), :], sp_out.at[pl.ds(0, 8)])
        pltpu.sync_copy(sp_out.at[pl.ds(0, 8)], out.at[pl.ds(C, 8), :])

    @pl.when((sid == 0) & (cid == 1))
    def _():
        pltpu.sync_copy(xnum.at[pl.ds(8, 5), :], sp_out.at[pl.ds(0, 5)])
        pltpu.sync_copy(sp_out.at[pl.ds(0, 5)], out.at[pl.ds(C + 8, 5), :])


_sc_call = pl.kernel(
    _sc_body,
    out_type=jax.ShapeDtypeStruct((OUT_H, B), jnp.float32),
    mesh=plsc.VectorSubcoreMesh(core_axis_name="c", subcore_axis_name="s"),
    compiler_params=pltpu.CompilerParams(
        use_tc_tiling_on_sc=True, needs_layout_passes=False
    ),
    scratch_types=[
        pltpu.VMEM_SHARED((D, B), jnp.float32),
        pltpu.VMEM_SHARED((2, B), jnp.int32),
        pltpu.VMEM((V,), jnp.float32),
        pltpu.VMEM((2, QB), jnp.int32),
        pltpu.VMEM((2, QB), jnp.float32),
        pltpu.SemaphoreType.DMA,
        pltpu.SemaphoreType.DMA,
        pltpu.SemaphoreType.DMA,
        pltpu.SemaphoreType.DMA,
    ],
)


@jax.jit
def kernel(x_categorical, x_numerical, tables):
    tab_t = jnp.transpose(tables, (0, 2, 1))
    xcat_t = jnp.transpose(x_categorical, (1, 0))
    xnum_t = jnp.transpose(x_numerical, (1, 0))
    out_t = _sc_call(tab_t, xcat_t, xnum_t)
    return jnp.transpose(out_t, (1, 0))


# speedup vs baseline: 1.1352x; 1.1352x over previous
import functools

import jax
import jax.numpy as jnp
from jax import lax
from jax.experimental import pallas as pl
from jax.experimental.pallas import tpu as pltpu
from jax.experimental.pallas import tpu_sc as plsc

B = 16384
F = 26
V = 100000
D = 16
NUM = 13
C = F * D
OUT_H = C + NUM

QB = B // 4
UNROLL = 4


def _sc_body(
    tab, xcat, xnum, out, sp_out, sp_idx, t_row, idx_v, out_v, sem_in, sem_out
):
    cid = lax.axis_index("c")
    sid = lax.axis_index("s")

    pltpu.async_copy(tab.at[cid, sid, :], t_row, sem_in)

    @pl.when(sid == 1)
    def _():
        pltpu.sync_copy(xcat.at[pl.ds(0, 2), :], sp_idx)

    out_desc = None
    for g in range(F // 2):
        f = 2 * g + cid

        pltpu.make_async_copy(tab.at[cid, sid, :], t_row, sem_in).wait()
        if out_desc is not None:
            @pl.when(sid == 0)
            def _():
                pltpu.make_async_copy(
                    sp_out, out.at[pl.ds(0, D), :], sem_out
                ).wait()

        plsc.subcore_barrier()

        for q in range(4):
            pltpu.sync_copy(sp_idx.at[cid, pl.ds(q * QB, QB)], idx_v)

            def gather_body(j, carry):
                for u in range(UNROLL):
                    o = j * (16 * UNROLL) + u * 16
                    iv = idx_v[pl.ds(o, 16)]
                    out_v[pl.ds(o, 16)] = plsc.load_gather(t_row, [iv])
                return carry

            lax.fori_loop(0, QB // (16 * UNROLL), gather_body, 0)

            pltpu.sync_copy(out_v, sp_out.at[sid, pl.ds(q * QB, QB)])

        if g + 1 < F // 2:
            pltpu.async_copy(tab.at[2 * (g + 1) + cid, sid, :], t_row, sem_in)

        plsc.subcore_barrier()

        if g + 1 < F // 2:
            @pl.when(sid == 1)
            def _():
                pltpu.sync_copy(xcat.at[pl.ds(2 * (g + 1), 2), :], sp_idx)

        @pl.when(sid == 0)
        def _():
            pltpu.async_copy(sp_out, out.at[pl.ds(f * D, D), :], sem_out)
        out_desc = True

    @pl.when(sid == 0)
    def _():
        pltpu.make_async_copy(sp_out, out.at[pl.ds(0, D), :], sem_out).wait()

    @pl.when((sid == 0) & (cid == 0))
    def _():
        pltpu.sync_copy(xnum.at[pl.ds(0, 8), :], sp_out.at[pl.ds(0, 8)])
        pltpu.sync_copy(sp_out.at[pl.ds(0, 8)], out.at[pl.ds(C, 8), :])

    @pl.when((sid == 0) & (cid == 1))
    def _():
        pltpu.sync_copy(xnum.at[pl.ds(8, 5), :], sp_out.at[pl.ds(0, 5)])
        pltpu.sync_copy(sp_out.at[pl.ds(0, 5)], out.at[pl.ds(C + 8, 5), :])


_sc_call = pl.kernel(
    _sc_body,
    out_type=jax.ShapeDtypeStruct((OUT_H, B), jnp.float32),
    mesh=plsc.VectorSubcoreMesh(core_axis_name="c", subcore_axis_name="s"),
    compiler_params=pltpu.CompilerParams(
        use_tc_tiling_on_sc=True, needs_layout_passes=False
    ),
    scratch_types=[
        pltpu.VMEM_SHARED((D, B), jnp.float32),
        pltpu.VMEM_SHARED((2, B), jnp.int32),
        pltpu.VMEM((V,), jnp.float32),
        pltpu.VMEM((QB,), jnp.int32),
        pltpu.VMEM((QB,), jnp.float32),
        pltpu.SemaphoreType.DMA,
        pltpu.SemaphoreType.DMA,
    ],
)


@jax.jit
def kernel(x_categorical, x_numerical, tables):
    tab_t = jnp.transpose(tables, (0, 2, 1))
    xcat_t = jnp.transpose(x_categorical, (1, 0))
    xnum_t = jnp.transpose(x_numerical, (1, 0))
    out_t = _sc_call(tab_t, xcat_t, xnum_t)
    return jnp.transpose(out_t, (1, 0))
